# baseline (speedup 1.0000x reference)
import jax
import jax.numpy as jnp
from jax import lax
from jax.experimental import pallas as pl
from jax.experimental.pallas import tpu as pltpu
from jax.experimental.pallas import tpu_sc as plsc

_K = 8
_T = 1024
_NW = 32
_LANES = 16


def _matmul_kernel(x_ref, wt_ref, bias_ref, logits_ref):
    x = x_ref[...]
    wt = wt_ref[...]
    logits_ref[...] = jax.lax.dot_general(
        wt, x, (((1,), (1,)), ((), ())),
        preferred_element_type=jnp.float32) + bias_ref[...]


def _sc_router(logits_hbm, biasx_hbm, sel_hbm, probs_hbm,
               lbuf, selbuf, pbuf, rawbuf, biasx_v):
    c = lax.axis_index("c")
    s = lax.axis_index("s")
    wid = s * 2 + c
    chunk = lbuf.shape[1]
    L = lbuf.shape[0]
    base = wid * chunk

    pltpu.sync_copy(logits_hbm.at[:, pl.ds(base, chunk)], lbuf)
    pltpu.sync_copy(biasx_hbm, biasx_v)

    neg_inf = jnp.float32(-jnp.inf)

    def group_body(g, carry):
        tok = pl.ds(g * _LANES, _LANES)
        idx = jnp.full((_LANES,), -1, jnp.int32)
        for k in range(_K):
            m = jnp.full((_LANES,), neg_inf, jnp.float32)
            if k == 0:
                for e in range(L):
                    m = jnp.maximum(m, lbuf[e, tok])
            else:
                for e in range(L):
                    v = jnp.where(idx == e, neg_inf, lbuf[e, tok])
                    lbuf[e, tok] = v
                    m = jnp.maximum(m, v)
            nidx = jnp.full((_LANES,), L, jnp.int32)
            bias_sel = jnp.zeros((_LANES,), jnp.float32)
            for e in range(L - 1, -1, -1):
                eq = lbuf[e, tok] == m
                nidx = jnp.where(eq, jnp.int32(e), nidx)
                bias_sel = jnp.where(eq, biasx_v[e, :], bias_sel)
            idx = nidx
            selbuf[k, tok] = idx
            rawbuf[pl.ds(k * _LANES, _LANES)] = m - bias_sel
        mx = jnp.full((_LANES,), neg_inf, jnp.float32)
        for k in range(_K):
            mx = jnp.maximum(mx, rawbuf[pl.ds(k * _LANES, _LANES)])
        tot = jnp.zeros((_LANES,), jnp.float32)
        es = []
        for k in range(_K):
            e_k = jnp.exp(rawbuf[pl.ds(k * _LANES, _LANES)] - mx)
            es.append(e_k)
            tot = tot + e_k
        for k in range(_K):
            pbuf[k, tok] = es[k] / tot
        return carry

    lax.fori_loop(0, chunk // _LANES, group_body, 0)

    pltpu.sync_copy(selbuf, sel_hbm.at[:, pl.ds(base, chunk)])
    pltpu.sync_copy(pbuf, probs_hbm.at[:, pl.ds(base, chunk)])


def _epilogue_kernel(sel_ref, act_ref, bias_ref, loss_ref, vio_ref,
                     counts_scr, act_scr):
    i = pl.program_id(0)
    nsteps = pl.num_programs(0)

    @pl.when(i == 0)
    def _init():
        counts_scr[...] = jnp.zeros_like(counts_scr)
        act_scr[...] = jnp.zeros_like(act_scr)

    sel = sel_ref[...]
    act = act_ref[...]
    L = counts_scr.shape[0]
    C = sel.shape[1]
    iota = jax.lax.broadcasted_iota(jnp.int32, (L, C), 0)
    onehot_sum = jnp.zeros((L, C), jnp.float32)
    for k in range(sel.shape[0]):
        onehot_sum += (iota == sel[k:k + 1, :]).astype(jnp.float32)
    counts_scr[...] += jnp.sum(onehot_sum * act, axis=1, keepdims=True)
    act_scr[...] += jnp.sum(act, axis=(0, 1), keepdims=True)

    @pl.when(i == nsteps - 1)
    def _finish():
        counts = counts_scr[...]
        total = act_scr[...] * jnp.float32(_K)
        freqs = counts / total
        bias = bias_ref[...]
        loss_ref[...] = jnp.sum(bias * freqs, axis=0, keepdims=True)
        vio_ref[...] = jnp.float32(L) * jnp.max(freqs - 1.0 / L, axis=0,
                                                keepdims=True)


def kernel(x, active_mask, W_r, expert_bias):
    Bb, Nn, Hh = x.shape
    L = W_r.shape[1]
    BN = Bb * Nn
    chunk = BN // _NW
    xf = x.reshape(BN, Hh)
    wt = W_r.T
    actf = active_mask.reshape(1, BN).astype(jnp.float32)

    logits = pl.pallas_call(
        _matmul_kernel,
        grid=(BN // _T,),
        in_specs=[
            pl.BlockSpec((_T, Hh), lambda i: (i, 0)),
            pl.BlockSpec((L, Hh), lambda i: (0, 0)),
            pl.BlockSpec((L, 1), lambda i: (0, 0)),
        ],
        out_specs=pl.BlockSpec((L, _T), lambda i: (0, i)),
        out_shape=jax.ShapeDtypeStruct((L, BN), jnp.float32),
    )(xf, wt, expert_bias.reshape(L, 1))

    sc = pl.kernel(
        _sc_router,
        mesh=plsc.VectorSubcoreMesh(core_axis_name="c", subcore_axis_name="s"),
        out_type=[
            jax.ShapeDtypeStruct((_K, BN), jnp.int32),
            jax.ShapeDtypeStruct((_K, BN), jnp.float32),
        ],
        scratch_types=[
            pltpu.VMEM((L, chunk), jnp.float32),
            pltpu.VMEM((_K, chunk), jnp.int32),
            pltpu.VMEM((_K, chunk), jnp.float32),
            pltpu.VMEM((_K * _LANES,), jnp.float32),
            pltpu.VMEM((L, _LANES), jnp.float32),
        ],
    )
    biasx = jnp.broadcast_to(expert_bias[:, None], (L, _LANES))
    sel, probs = sc(logits, biasx)

    CE = 2048
    loss, vio = pl.pallas_call(
        _epilogue_kernel,
        grid=(BN // CE,),
        in_specs=[
            pl.BlockSpec((_K, CE), lambda i: (0, i)),
            pl.BlockSpec((1, CE), lambda i: (0, i)),
            pl.BlockSpec((L, 1), lambda i: (0, 0)),
        ],
        out_specs=[
            pl.BlockSpec((1, 1), lambda i: (0, 0)),
            pl.BlockSpec((1, 1), lambda i: (0, 0)),
        ],
        out_shape=[
            jax.ShapeDtypeStruct((1, 1), jnp.float32),
            jax.ShapeDtypeStruct((1, 1), jnp.float32),
        ],
        scratch_shapes=[
            pltpu.VMEM((L, 1), jnp.float32),
            pltpu.VMEM((1, 1), jnp.float32),
        ],
    )(sel, actf, expert_bias.reshape(L, 1))

    return (sel.T.reshape(Bb, Nn, _K), probs.T.reshape(Bb, Nn, _K),
            loss[0, 0], vio[0, 0])


# speedup vs baseline: 1.8191x; 1.8191x over previous
import jax
import jax.numpy as jnp
from jax.experimental import pallas as pl
from jax.experimental.pallas import tpu as pltpu

_K = 8


def _router_kernel(x_ref, wt_ref, bias_ref, act_ref,
                   sel_ref, probs_ref, loss_ref, vio_ref,
                   counts_scr, act_scr):
    i = pl.program_id(0)
    nsteps = pl.num_programs(0)

    @pl.when(i == 0)
    def _init():
        counts_scr[...] = jnp.zeros_like(counts_scr)
        act_scr[...] = jnp.zeros_like(act_scr)

    x = x_ref[...]
    wt = wt_ref[...]
    logits = jax.lax.dot_general(wt, x, (((1,), (1,)), ((), ())),
                                 preferred_element_type=jnp.float32)
    bias = bias_ref[...]
    L, T = logits.shape
    iota = jax.lax.broadcasted_iota(jnp.int32, (L, T), 0)
    neg_inf = jnp.float32(-jnp.inf)

    b = logits + bias
    sel_rows = []
    val_rows = []
    for _ in range(_K):
        m = jnp.max(b, axis=0, keepdims=True)
        idx = jnp.min(jnp.where(b == m, iota, L), axis=0, keepdims=True)
        onehot = iota == idx
        val = jnp.max(jnp.where(onehot, logits, neg_inf), axis=0,
                      keepdims=True)
        sel_rows.append(idx)
        val_rows.append(val)
        b = jnp.where(onehot, neg_inf, b)
    onehot_sum = jnp.isneginf(b).astype(jnp.float32)

    sel = jnp.concatenate(sel_rows, axis=0)
    vals = jnp.concatenate(val_rows, axis=0)
    mx = jnp.max(vals, axis=0, keepdims=True)
    e = jnp.exp(vals - mx)
    probs = e / jnp.sum(e, axis=0, keepdims=True)

    sel_ref[...] = sel.T
    probs_ref[...] = probs.T

    act = act_ref[...]
    counts_scr[...] += jnp.sum(onehot_sum * act, axis=1, keepdims=True)
    act_scr[...] += jnp.sum(act, axis=(0, 1), keepdims=True)

    @pl.when(i == nsteps - 1)
    def _finish():
        counts = counts_scr[...]
        total = act_scr[...] * jnp.float32(_K)
        freqs = counts / total
        loss_ref[...] = jnp.sum(bias * freqs, axis=0, keepdims=True)
        vio_ref[...] = jnp.float32(L) * jnp.max(freqs - 1.0 / L, axis=0,
                                                keepdims=True)


def kernel(x, active_mask, W_r, expert_bias):
    Bb, Nn, Hh = x.shape
    L = W_r.shape[1]
    BN = Bb * Nn
    T = 1024
    xf = x.reshape(BN, Hh)
    wt = W_r.T
    act = active_mask.reshape(1, BN).astype(jnp.float32)
    bias2 = expert_bias.reshape(L, 1)

    out_shape = [
        jax.ShapeDtypeStruct((BN, _K), jnp.int32),
        jax.ShapeDtypeStruct((BN, _K), jnp.float32),
        jax.ShapeDtypeStruct((1, 1), jnp.float32),
        jax.ShapeDtypeStruct((1, 1), jnp.float32),
    ]
    sel, probs, loss, vio = pl.pallas_call(
        _router_kernel,
        grid=(BN // T,),
        in_specs=[
            pl.BlockSpec((T, Hh), lambda i: (i, 0)),
            pl.BlockSpec((L, Hh), lambda i: (0, 0)),
            pl.BlockSpec((L, 1), lambda i: (0, 0)),
            pl.BlockSpec((1, T), lambda i: (0, i)),
        ],
        out_specs=[
            pl.BlockSpec((T, _K), lambda i: (i, 0)),
            pl.BlockSpec((T, _K), lambda i: (i, 0)),
            pl.BlockSpec((1, 1), lambda i: (0, 0)),
            pl.BlockSpec((1, 1), lambda i: (0, 0)),
        ],
        out_shape=out_shape,
        scratch_shapes=[
            pltpu.VMEM((L, 1), jnp.float32),
            pltpu.VMEM((1, 1), jnp.float32),
        ],
    )(xf, wt, bias2, act)

    return (sel.reshape(Bb, Nn, _K), probs.reshape(Bb, Nn, _K),
            loss[0, 0], vio[0, 0])


# speedup vs baseline: 2.1628x; 1.1889x over previous
import jax
import jax.numpy as jnp
from jax.experimental import pallas as pl
from jax.experimental.pallas import tpu as pltpu

_K = 8


def _router_kernel(x_ref, wt_ref, bias_ref, act_ref,
                   sel_ref, probs_ref, loss_ref, vio_ref,
                   counts_scr, act_scr):
    i = pl.program_id(0)
    nsteps = pl.num_programs(0)

    @pl.when(i == 0)
    def _init():
        counts_scr[...] = jnp.zeros_like(counts_scr)
        act_scr[...] = jnp.zeros_like(act_scr)

    x = x_ref[...]
    wt = wt_ref[...]
    logits = jax.lax.dot_general(wt, x, (((1,), (1,)), ((), ())),
                                 preferred_element_type=jnp.float32)
    bias = bias_ref[...]
    L, T = logits.shape
    iota = jax.lax.broadcasted_iota(jnp.int32, (L, T), 0)
    neg_inf = jnp.float32(-jnp.inf)

    b = logits + bias
    sel_rows = []
    val_rows = []
    for _ in range(_K):
        m = jnp.max(b, axis=0, keepdims=True)
        idx = jnp.min(jnp.where(b == m, iota, L), axis=0, keepdims=True)
        onehot = iota == idx
        val = jnp.max(jnp.where(onehot, logits, neg_inf), axis=0,
                      keepdims=True)
        sel_rows.append(idx)
        val_rows.append(val)
        b = jnp.where(onehot, neg_inf, b)
    onehot_sum = jnp.isneginf(b).astype(jnp.float32)

    sel = jnp.concatenate(sel_rows, axis=0)
    vals = jnp.concatenate(val_rows, axis=0)
    mx = jnp.max(vals, axis=0, keepdims=True)
    e = jnp.exp(vals - mx)
    probs = e / jnp.sum(e, axis=0, keepdims=True)

    sel_ref[...] = sel
    probs_ref[...] = probs

    act = act_ref[...]
    counts_scr[...] += jnp.sum(onehot_sum * act, axis=1, keepdims=True)
    act_scr[...] += jnp.sum(act, axis=(0, 1), keepdims=True)

    @pl.when(i == nsteps - 1)
    def _finish():
        counts = counts_scr[...]
        total = act_scr[...] * jnp.float32(_K)
        freqs = counts / total
        loss_ref[...] = jnp.sum(bias * freqs, axis=0, keepdims=True)
        vio_ref[...] = jnp.float32(L) * jnp.max(freqs - 1.0 / L, axis=0,
                                                keepdims=True)


def kernel(x, active_mask, W_r, expert_bias):
    Bb, Nn, Hh = x.shape
    L = W_r.shape[1]
    BN = Bb * Nn
    T = 1024
    xf = x.reshape(BN, Hh)
    wt = W_r.T
    act = active_mask.reshape(1, BN).astype(jnp.float32)
    bias2 = expert_bias.reshape(L, 1)

    out_shape = [
        jax.ShapeDtypeStruct((_K, BN), jnp.int32),
        jax.ShapeDtypeStruct((_K, BN), jnp.float32),
        jax.ShapeDtypeStruct((1, 1), jnp.float32),
        jax.ShapeDtypeStruct((1, 1), jnp.float32),
    ]
    sel, probs, loss, vio = pl.pallas_call(
        _router_kernel,
        grid=(BN // T,),
        in_specs=[
            pl.BlockSpec((T, Hh), lambda i: (i, 0)),
            pl.BlockSpec((L, Hh), lambda i: (0, 0)),
            pl.BlockSpec((L, 1), lambda i: (0, 0)),
            pl.BlockSpec((1, T), lambda i: (0, i)),
        ],
        out_specs=[
            pl.BlockSpec((_K, T), lambda i: (0, i)),
            pl.BlockSpec((_K, T), lambda i: (0, i)),
            pl.BlockSpec((1, 1), lambda i: (0, 0)),
            pl.BlockSpec((1, 1), lambda i: (0, 0)),
        ],
        out_shape=out_shape,
        scratch_shapes=[
            pltpu.VMEM((L, 1), jnp.float32),
            pltpu.VMEM((1, 1), jnp.float32),
        ],
    )(xf, wt, bias2, act)

    return (sel.T.reshape(Bb, Nn, _K), probs.T.reshape(Bb, Nn, _K),
            loss[0, 0], vio[0, 0])
